# baseline (speedup 1.0000x reference)
import functools

import jax
import jax.numpy as jnp
from jax import lax
from jax.experimental import pallas as pl
from jax.experimental.pallas import tpu as pltpu
from jax.experimental.pallas import tpu_sc as plsc

N_FIELDS = 26
VOCAB = 100000
DIM = 32
BATCH = 16384

NC, NS, L = 2, 16, 16
NW = NC * NS
ROWS = BATCH * N_FIELDS
ROWS_PER_W = ROWS // NW
CHUNK = 1664
N_CHUNKS = ROWS_PER_W // CHUNK


def _make_sc_gather():
    mesh = plsc.VectorSubcoreMesh(core_axis_name="c", subcore_axis_name="s")

    @functools.partial(
        pl.kernel,
        mesh=mesh,
        out_type=jax.ShapeDtypeStruct((ROWS, DIM), jnp.float32),
        compiler_params=pltpu.CompilerParams(use_tc_tiling_on_sc=False),
        scratch_types=[
            pltpu.VMEM((CHUNK,), jnp.int32),
            pltpu.VMEM((CHUNK,), jnp.int32),
            pltpu.VMEM((CHUNK, DIM), jnp.float32),
            pltpu.SemaphoreType.DMA,
        ],
    )
    def gather_kernel(table_hbm, xflat_hbm, off_hbm, out_hbm,
                      idx_v, off_v, rows_v, sem):
        wid = lax.axis_index("s") * NC + lax.axis_index("c")
        base0 = wid * ROWS_PER_W
        pltpu.sync_copy(off_hbm, off_v)

        def do_chunk(c, carry):
            base = base0 + c * CHUNK
            pltpu.sync_copy(xflat_hbm.at[pl.ds(base, CHUNK)], idx_v)

            def add_off(s, carry2):
                sl = pl.ds(s * L, L)
                idx_v[sl] = idx_v[sl] + off_v[sl]
                return carry2

            lax.fori_loop(0, CHUNK // L, add_off, 0)
            pltpu.async_copy(table_hbm.at[idx_v], rows_v, sem).wait()
            pltpu.sync_copy(rows_v, out_hbm.at[pl.ds(base, CHUNK)])
            return carry

        lax.fori_loop(0, N_CHUNKS, do_chunk, 0)

    return gather_kernel


_sc_gather = _make_sc_gather()


def kernel(X, tables):
    table_flat = tables.reshape(N_FIELDS * VOCAB, DIM)
    x_flat = X.reshape(-1)
    offsets = jnp.tile(jnp.arange(N_FIELDS, dtype=jnp.int32) * VOCAB,
                       CHUNK // N_FIELDS)
    out = _sc_gather(table_flat, x_flat, offsets)
    return out.reshape(BATCH, N_FIELDS * DIM)


# speedup vs baseline: 3.1677x; 3.1677x over previous
import functools

import jax
import jax.numpy as jnp
from jax import lax
from jax.experimental import pallas as pl
from jax.experimental.pallas import tpu as pltpu
from jax.experimental.pallas import tpu_sc as plsc

N_FIELDS = 26
VOCAB = 100000
DIM = 32
BATCH = 16384

NC, NS, L = 2, 16, 16
NW = NC * NS
STRIPE = 4096


def _make_sc_gather():
    mesh = plsc.VectorSubcoreMesh(core_axis_name="c", subcore_axis_name="s")

    @functools.partial(
        pl.kernel,
        mesh=mesh,
        out_type=jax.ShapeDtypeStruct((N_FIELDS * DIM, BATCH), jnp.float32),
        compiler_params=pltpu.CompilerParams(needs_layout_passes=False),
        scratch_types=[
            pltpu.VMEM((VOCAB,), jnp.float32),
            pltpu.VMEM((STRIPE,), jnp.int32),
            pltpu.VMEM((STRIPE,), jnp.float32),
        ],
    )
    def gather_kernel(table_hbm, xt_hbm, out_hbm, plane_v, idx_v, out_v):
        d = lax.axis_index("s") * NC + lax.axis_index("c")

        def do_field(f, carry):
            pltpu.sync_copy(table_hbm.at[f, d], plane_v)
            p = f * DIM + d

            def do_stripe(s, carry2):
                pltpu.sync_copy(xt_hbm.at[f, pl.ds(s * STRIPE, STRIPE)], idx_v)

                def do_vec(i, carry3):
                    b = i * L
                    idx16 = idx_v[pl.ds(b, L)]
                    out_v[pl.ds(b, L)] = plsc.load_gather(plane_v, [idx16])
                    return carry3

                lax.fori_loop(0, STRIPE // L, do_vec, 0)
                pltpu.sync_copy(out_v, out_hbm.at[p, pl.ds(s * STRIPE, STRIPE)])
                return carry2

            lax.fori_loop(0, BATCH // STRIPE, do_stripe, 0)
            return carry

        lax.fori_loop(0, N_FIELDS, do_field, 0)

    return gather_kernel


_sc_gather = _make_sc_gather()


def kernel(X, tables):
    table_t = tables.transpose(0, 2, 1)
    x_t = X.T
    out_t = _sc_gather(table_t, x_t)
    return out_t.T.reshape(BATCH, N_FIELDS * DIM)
